# baseline (speedup 1.0000x reference)
import functools

import jax
import jax.numpy as jnp
from jax import lax
from jax.experimental import pallas as pl
from jax.experimental.pallas import tpu as pltpu
from jax.experimental.pallas import tpu_sc as plsc

BATCH = 4096
NUM_SLOTS = 520
VOCAB = 1000000
EMB = 16

NUM_WORKERS = 32
ROWS_PER_WORKER = BATCH // NUM_WORKERS
CHUNK_ROWS = 4
CHUNK_SLOTS = CHUNK_ROWS * NUM_SLOTS
PAIR_SLOTS = 2 * NUM_SLOTS
SUPERCHUNKS = ROWS_PER_WORKER // 16

PACK_ROWS = VOCAB // 8
PACK_BLK = 1000
HI_MASK = jnp.int32(-65536)


def _pack_body(w_ref, v_ref, out_ref):
    v = v_ref[...]
    w = w_ref[...]
    vb = v.astype(jnp.bfloat16)
    ex = jnp.broadcast_to(w[:, :, None], (PACK_BLK, 8, 16))
    ex = ex.reshape(PACK_BLK, 128)
    lane = lax.broadcasted_iota(jnp.int32, (PACK_BLK, 128), 1)
    exb = jnp.where(lane % 16 == 0, ex, 0.0).astype(jnp.bfloat16)
    lo = lax.bitcast_convert_type(vb, jnp.uint16).astype(jnp.int32)
    hi = lax.bitcast_convert_type(exb, jnp.uint16).astype(jnp.int32)
    out_ref[...] = (hi << 16) | lo


@jax.jit
def _pack_tables(w_first, v_second):
    v128 = v_second.reshape(PACK_ROWS, 128)
    w8 = w_first.reshape(PACK_ROWS, 8)
    out = pl.pallas_call(
        _pack_body,
        grid=(PACK_ROWS // PACK_BLK,),
        in_specs=[
            pl.BlockSpec((PACK_BLK, 8), lambda i: (i, 0)),
            pl.BlockSpec((PACK_BLK, 128), lambda i: (i, 0)),
        ],
        out_specs=pl.BlockSpec((PACK_BLK, 128), lambda i: (i, 0)),
        out_shape=jax.ShapeDtypeStruct((PACK_ROWS, 128), jnp.int32),
    )(w8, v128)
    return out.reshape(VOCAB, EMB)


def _fm_body(vals_hbm, comb_hbm, idx_hbm, out_hbm,
             idx_v0, idx_v1, val_v0, val_v1,
             vrows_v0, vrows_v1, out_v,
             sem_v0, sem_v1):
    num_cores = 2
    wid = lax.axis_index("s") * num_cores + lax.axis_index("c")
    lane_iota = lax.iota(jnp.int32, 16)

    bufs = [
        (idx_v0, val_v0, vrows_v0, sem_v0),
        (idx_v1, val_v1, vrows_v1, sem_v1),
    ]

    def fire(gc, b):
        idx_b, val_b, vr_b, sv = bufs[b]
        base = wid * ROWS_PER_WORKER * NUM_SLOTS + gc * CHUNK_SLOTS
        pltpu.sync_copy(idx_hbm.at[pl.ds(base, CHUNK_SLOTS)], idx_b)
        pltpu.sync_copy(vals_hbm.at[pl.ds(base, CHUNK_SLOTS)], val_b)
        pltpu.async_copy(comb_hbm.at[idx_b], vr_b, sv)

    def drain(b):
        _, _, vr_b, sv = bufs[b]
        pltpu.make_async_copy(
            comb_hbm.at[pl.ds(0, CHUNK_SLOTS)], vr_b, sv).wait()

    def lane_sum(x):
        for sh in (8, 4, 2, 1):
            perm = lane_iota ^ sh
            x = x + x.at[perm].get(mode="promise_in_bounds")
        return x

    z = jnp.zeros((16,), jnp.float32)

    def slot_update(row_u, valk, acc, sq, ex):
        va = plsc.bitcast(row_u << 16, jnp.float32)
        ea = plsc.bitcast(row_u & HI_MASK, jnp.float32)
        t = va * valk
        acc = acc + t
        sq = sq + t * t
        ex = ex + ea * valk
        return acc, sq, ex

    fire(0, 0)

    def superchunk_body(sc, _):
        outvec = jnp.zeros((16,), jnp.float32)
        for sub in range(4):
            p = sub % 2
            _, val_b, vr_b, _ = bufs[p]
            gc = sc * 4 + sub
            drain(p)
            if sub < 3:
                fire(gc + 1, 1 - p)
            else:
                @pl.when(sc < SUPERCHUNKS - 1)
                def _():
                    fire(gc + 1, 1 - p)

            def half_row(base, carry0):
                acc0, sq0, ex0 = carry0

                def group(g, carry):
                    a0, a1, q0, q1, e0, e1 = carry
                    s0 = base + g * 16
                    valvec = val_b[pl.ds(s0, 16)]
                    accs = [a0, a1]
                    sqs = [q0, q1]
                    exs = [e0, e1]
                    for k in range(16):
                        j = k % 2
                        accs[j], sqs[j], exs[j] = slot_update(
                            vr_b[s0 + k, :], valvec[k],
                            accs[j], sqs[j], exs[j])
                    return (*accs, *sqs, *exs)

                a0, a1, q0, q1, e0, e1 = lax.fori_loop(
                    0, 32, group, (acc0, z, sq0, z, ex0, z))
                return a0 + a1, q0 + q1, e0 + e1

            for pair in range(2):
                pbase = pair * PAIR_SLOTS
                accA, sqA, exA = half_row(pbase, (z, z, z))
                accB, sqB, exB = z, z, z
                sm = pbase + 512
                valvec = val_b[pl.ds(sm, 16)]
                for k in range(16):
                    if k < 8:
                        accA, sqA, exA = slot_update(
                            vr_b[sm + k, :], valvec[k], accA, sqA, exA)
                    else:
                        accB, sqB, exB = slot_update(
                            vr_b[sm + k, :], valvec[k], accB, sqB, exB)
                accB, sqB, exB = half_row(pbase + 528, (accB, sqB, exB))

                for (acc, sq, ex, lane) in (
                        (accA, sqA, exA, sub * 4 + pair * 2),
                        (accB, sqB, exB, sub * 4 + pair * 2 + 1)):
                    combined = 0.5 * (acc * acc - sq) + ex
                    total = lane_sum(combined)
                    outvec = jnp.where(lane_iota == lane, total, outvec)
        out_v[pl.ds(sc * 16, 16)] = outvec
        return 0

    lax.fori_loop(0, SUPERCHUNKS, superchunk_body, 0)
    pltpu.sync_copy(out_v, out_hbm.at[pl.ds(wid * ROWS_PER_WORKER,
                                            ROWS_PER_WORKER)])


@jax.jit
def _fm_sc(vals_flat, comb, idx_flat):
    mesh = plsc.VectorSubcoreMesh(core_axis_name="c", subcore_axis_name="s")
    return pl.kernel(
        _fm_body,
        out_type=jax.ShapeDtypeStruct((BATCH,), jnp.float32),
        mesh=mesh,
        compiler_params=pltpu.CompilerParams(use_tc_tiling_on_sc=False,
                                             needs_layout_passes=False),
        scratch_types=[
            pltpu.VMEM((CHUNK_SLOTS,), jnp.int32),
            pltpu.VMEM((CHUNK_SLOTS,), jnp.int32),
            pltpu.VMEM((CHUNK_SLOTS,), jnp.float32),
            pltpu.VMEM((CHUNK_SLOTS,), jnp.float32),
            pltpu.VMEM((CHUNK_SLOTS, EMB), jnp.int32),
            pltpu.VMEM((CHUNK_SLOTS, EMB), jnp.int32),
            pltpu.VMEM((ROWS_PER_WORKER,), jnp.float32),
            pltpu.SemaphoreType.DMA,
            pltpu.SemaphoreType.DMA,
        ],
    )(vals_flat, comb, idx_flat)


def kernel(feature_values, w_first, v_second, fm_bias, feature_idx):
    idx_flat = feature_idx.astype(jnp.int32).reshape(-1)
    vals_flat = feature_values.reshape(-1)
    comb = _pack_tables(w_first, v_second)
    logits = _fm_sc(vals_flat, comb, idx_flat)
    return logits + fm_bias[0]
